# baseline (speedup 1.0000x reference)
import functools

import jax
import jax.numpy as jnp
from jax import lax
from jax.experimental import pallas as pl
from jax.experimental.pallas import tpu as pltpu
from jax.experimental.pallas import tpu_sc as plsc

NC = 2
NS = 16
LN = 16
TM = 256


def _splat(vec, l):
    return lax.gather(
        vec, jnp.full((LN, 1), l, jnp.int32),
        lax.GatherDimensionNumbers(
            offset_dims=(), collapsed_slice_dims=(0,), start_index_map=(0,)),
        (1,), mode=lax.GatherScatterMode.PROMISE_IN_BOUNDS)


def _gate_body(x_ref, gwt_ref, gb_ref, eid_ref, w_ref):
    x = x_ref[...]
    logits = lax.dot_general(
        x, gwt_ref[...], (((1,), (0,)), ((), ())),
        preferred_element_type=jnp.float32) + gb_ref[...]
    ne = logits.shape[1]
    lanes = lax.broadcasted_iota(jnp.int32, logits.shape, 1)
    m1 = jnp.max(logits, axis=1, keepdims=True)
    idx1 = jnp.min(jnp.where(logits == m1, lanes, ne), axis=1, keepdims=True)
    l2 = jnp.where(lanes == idx1, -1e30, logits)
    m2 = jnp.max(l2, axis=1, keepdims=True)
    idx2 = jnp.min(jnp.where(l2 == m2, lanes, ne), axis=1, keepdims=True)
    e21 = jnp.exp(m2 - m1)
    w1 = 1.0 / (1.0 + e21)
    w2 = e21 / (1.0 + e21)
    eid_ref[...] = jnp.concatenate([idx1, idx2], axis=1)
    w_ref[...] = jnp.concatenate([w1, w2], axis=1)


def _gate(inputs, gate_W, gate_b):
    t, d = inputs.shape
    ne = gate_W.shape[0]
    bm = 512
    return pl.pallas_call(
        _gate_body,
        grid=(t // bm,),
        in_specs=[
            pl.BlockSpec((bm, d), lambda m: (m, 0)),
            pl.BlockSpec((d, ne), lambda m: (0, 0)),
            pl.BlockSpec((1, ne), lambda m: (0, 0)),
        ],
        out_specs=[
            pl.BlockSpec((bm, 2), lambda m: (m, 0)),
            pl.BlockSpec((bm, 2), lambda m: (m, 0)),
        ],
        out_shape=[
            jax.ShapeDtypeStruct((t, 2), jnp.int32),
            jax.ShapeDtypeStruct((t, 2), jnp.float32),
        ],
        compiler_params=pltpu.CompilerParams(
            dimension_semantics=("parallel",)),
    )(inputs, gate_W.T, gate_b.reshape(1, ne))


def _routing(ek, wk, ne, pad_t, nt):
    tk = ek.shape[0]
    tmask = tk // 2 - 1
    pp = tk // NS
    ch = pp // LN
    mesh = plsc.VectorSubcoreMesh(
        core_axis_name="c", subcore_axis_name="s",
        num_cores=NC, num_subcores=NS)
    cp = pltpu.CompilerParams(needs_layout_passes=False)

    @functools.partial(
        pl.kernel,
        out_type=[
            jax.ShapeDtypeStruct((NS, LN), jnp.int32),
            jax.ShapeDtypeStruct((NS, pp), jnp.int32),
        ],
        mesh=mesh,
        compiler_params=cp,
        scratch_types=[
            pltpu.VMEM((pp,), jnp.int32),
            pltpu.VMEM((pp,), jnp.int32),
            pltpu.VMEM((LN,), jnp.int32),
        ],
    )
    def r1(ek_hbm, cnt_out, lr_out, ev, lr, crow):
        c = lax.axis_index("c")
        s = lax.axis_index("s")

        @pl.when(c == 0)
        def _():
            lane16 = lax.broadcasted_iota(jnp.int32, (LN,), 0)
            base = s * pp
            pltpu.sync_copy(ek_hbm.at[pl.ds(base, pp)], ev)
            zero16 = jnp.zeros((LN,), jnp.int32)

            def cbody(i, cnts):
                evc = ev[pl.ds(i * LN, LN)]
                lrv = zero16
                new = []
                for e in range(ne):
                    m = evc == e
                    r = plsc.cumsum(m.astype(jnp.int32))
                    lrv = lrv + jnp.where(m, cnts[e] + r - 1, 0)
                    new.append(cnts[e] + _splat(r, LN - 1))
                lr[pl.ds(i * LN, LN)] = lrv
                return tuple(new)
            cnts = lax.fori_loop(
                0, ch, cbody, tuple(zero16 for _ in range(ne)))
            cv = zero16
            for e in range(ne):
                cv = cv + jnp.where(lane16 == e, cnts[e], 0)
            crow[pl.ds(0, LN)] = cv
            pltpu.sync_copy(crow, cnt_out.at[s])
            pltpu.sync_copy(lr, lr_out.at[s])

    cnt_tbl, lranks = r1(ek)

    @functools.partial(
        pl.kernel,
        out_type=[
            jax.ShapeDtypeStruct((NS, pp), jnp.int32),
            jax.ShapeDtypeStruct((80,), jnp.int32),
        ],
        mesh=mesh,
        compiler_params=cp,
        scratch_types=[
            pltpu.VMEM((pp,), jnp.int32),
            pltpu.VMEM((pp,), jnp.int32),
            pltpu.VMEM((pp,), jnp.int32),
            pltpu.VMEM((NS, LN), jnp.int32),
            pltpu.VMEM((80,), jnp.int32),
        ],
    )
    def r2(ek_hbm, cnt_hbm, lr_hbm, pos_out, eot_out,
           ev, lr, posb, tblv, eotb):
        c = lax.axis_index("c")
        s = lax.axis_index("s")

        @pl.when(c == 0)
        def _():
            base = s * pp
            pltpu.sync_copy(ek_hbm.at[pl.ds(base, pp)], ev)
            pltpu.sync_copy(lr_hbm.at[s], lr)
            pltpu.sync_copy(cnt_hbm, tblv)
            zero16 = jnp.zeros((LN,), jnp.int32)

            def rbody(ti, carry):
                tot, pre = carry
                row = tblv[ti]
                return tot + row, pre + row * (ti < s).astype(jnp.int32)
            tot, pre = lax.fori_loop(0, NS, rbody, (zero16, zero16))
            rounded = ((tot + (TM - 1)) >> 8) << 8
            incl = plsc.cumsum(rounded)
            startv = incl - rounded
            mybase = startv + pre
            mb = tuple(_splat(mybase, e) for e in range(ne))

            def cbody(i, _):
                evc = ev[pl.ds(i * LN, LN)]
                lrv = lr[pl.ds(i * LN, LN)]
                posv = zero16
                for e in range(ne):
                    posv = posv + jnp.where(evc == e, mb[e] + lrv, 0)
                posb[pl.ds(i * LN, LN)] = posv
                return 0
            lax.fori_loop(0, ch, cbody, 0)
            pltpu.sync_copy(posb, pos_out.at[s])

            @pl.when(s == 0)
            def _eot():
                lane16 = lax.broadcasted_iota(jnp.int32, (LN,), 0)
                inls = tuple(_splat(incl, e) for e in range(ne))

                def ebody(ci, _):
                    tvec = (lane16 + ci * LN) * TM
                    acc = jnp.zeros((LN,), jnp.int32)
                    for e in range(ne):
                        acc = acc + (tvec >= inls[e]).astype(jnp.int32)
                    eotb[pl.ds(ci * LN, LN)] = jnp.minimum(acc, ne - 1)
                    return 0
                lax.fori_loop(0, 80 // LN, ebody, 0)
                pltpu.sync_copy(eotb, eot_out)

    pos2d, eot = r2(ek, cnt_tbl, lranks)

    @functools.partial(
        pl.kernel,
        out_type=[
            jax.ShapeDtypeStruct((pad_t,), jnp.int32),
            jax.ShapeDtypeStruct((pad_t,), jnp.float32),
        ],
        mesh=mesh,
        compiler_params=cp,
        scratch_types=[
            pltpu.VMEM((tk,), jnp.int32),
            pltpu.VMEM((tk,), jnp.float32),
            pltpu.VMEM((pad_t,), jnp.int32),
            pltpu.VMEM((pad_t,), jnp.float32),
        ],
    )
    def r3(pos_hbm, wk_hbm, tok_out, w_out, pv, wv, tokb, wsb):
        c = lax.axis_index("c")
        s = lax.axis_index("s")

        @pl.when(jnp.logical_and(c == 0, s == 0))
        def _():
            lane16 = lax.broadcasted_iota(jnp.int32, (LN,), 0)
            pltpu.sync_copy(pos_hbm, pv)
            pltpu.sync_copy(wk_hbm, wv)

            def zb(i, _):
                tokb[pl.ds(i * LN, LN)] = jnp.zeros((LN,), jnp.int32)
                wsb[pl.ds(i * LN, LN)] = jnp.zeros((LN,), jnp.float32)
                return 0
            lax.fori_loop(0, pad_t // LN, zb, 0)

            def cb(i, _):
                posv = pv[pl.ds(i * LN, LN)]
                tokv = (jnp.full((LN,), i * LN, jnp.int32)
                        + lane16) & tmask
                plsc.store_scatter(tokb, [posv], tokv)
                plsc.store_scatter(wsb, [posv], wv[pl.ds(i * LN, LN)])
                return 0
            lax.fori_loop(0, tk // LN, cb, 0)
            pltpu.sync_copy(tokb, tok_out)
            pltpu.sync_copy(wsb, w_out)

    tok_sorted, w_sorted = r3(pos2d.reshape(-1), wk)
    return tok_sorted, w_sorted, pos2d.reshape(-1), eot


def _gather(inputs, tok3, pad_t):
    t, d = inputs.shape
    nw = NC * NS
    rows_w = pad_t // nw
    cr = 32
    nch = rows_w // cr
    mesh = plsc.VectorSubcoreMesh(
        core_axis_name="c", subcore_axis_name="s",
        num_cores=NC, num_subcores=NS)

    @functools.partial(
        pl.kernel,
        out_type=jax.ShapeDtypeStruct((pad_t, d), jnp.float32),
        mesh=mesh,
        compiler_params=pltpu.CompilerParams(needs_layout_passes=False),
        scratch_types=[
            pltpu.VMEM((nch, cr), jnp.int32),
            pltpu.VMEM((2, cr, d), jnp.float32),
            pltpu.SemaphoreType.DMA,
            pltpu.SemaphoreType.DMA,
            pltpu.SemaphoreType.DMA,
            pltpu.SemaphoreType.DMA,
        ],
    )
    def k(x_hbm, tok_hbm, xs_hbm, idx_v, bufs, g0, g1, w0, w1):
        c = lax.axis_index("c")
        s = lax.axis_index("s")
        wid = s * NC + c
        pltpu.sync_copy(tok_hbm.at[wid], idx_v)
        base = wid * rows_w
        gsem = (g0, g1)
        wsem = (w0, w1)
        gops = [None] * nch
        wops = [None] * nch
        gops[0] = pltpu.async_copy(
            x_hbm.at[idx_v.at[0]], bufs.at[0], gsem[0])
        for ci in range(nch):
            b = ci & 1
            if ci + 1 < nch:
                if ci >= 1:
                    wops[ci - 1].wait()
                gops[ci + 1] = pltpu.async_copy(
                    x_hbm.at[idx_v.at[ci + 1]], bufs.at[1 - b],
                    gsem[1 - b])
            gops[ci].wait()
            wops[ci] = pltpu.async_copy(
                bufs.at[b], xs_hbm.at[pl.ds(base + ci * cr, cr)], wsem[b])
        wops[nch - 2].wait()
        wops[nch - 1].wait()

    return k(inputs, tok3)


def _mm_body(eot_ref, xs_ref, ws_ref, ew_ref, eb_ref, y_ref):
    x = xs_ref[...]
    acc = lax.dot_general(
        x, ew_ref[0], (((1,), (1,)), ((), ())),
        preferred_element_type=jnp.float32)
    y_ref[...] = ws_ref[...] * (acc + eb_ref[0])


def _grouped_mm(eot, xs, ws, expert_W, expert_b, nt):
    ne, d, _ = expert_W.shape
    grid_spec = pltpu.PrefetchScalarGridSpec(
        num_scalar_prefetch=1,
        grid=(nt,),
        in_specs=[
            pl.BlockSpec((TM, d), lambda i, eot: (i, 0)),
            pl.BlockSpec((TM, 1), lambda i, eot: (i, 0)),
            pl.BlockSpec((1, d, d), lambda i, eot: (eot[i], 0, 0)),
            pl.BlockSpec((1, 1, d), lambda i, eot: (eot[i], 0, 0)),
        ],
        out_specs=pl.BlockSpec((TM, d), lambda i, eot: (i, 0)),
    )
    return pl.pallas_call(
        _mm_body,
        grid_spec=grid_spec,
        out_shape=jax.ShapeDtypeStruct((nt * TM, d), jnp.float32),
        compiler_params=pltpu.CompilerParams(
            dimension_semantics=("arbitrary",)),
    )(eot, xs, ws.reshape(-1, 1), expert_W, expert_b.reshape(ne, 1, d))


def _combine(y, pos0, pos1, t, d):
    nw = NC * NS
    tok_w = t // nw
    ct = 16
    nch = tok_w // ct
    mesh = plsc.VectorSubcoreMesh(
        core_axis_name="c", subcore_axis_name="s",
        num_cores=NC, num_subcores=NS)

    @functools.partial(
        pl.kernel,
        out_type=jax.ShapeDtypeStruct((t, d), jnp.float32),
        mesh=mesh,
        compiler_params=pltpu.CompilerParams(needs_layout_passes=False),
        scratch_types=[
            pltpu.VMEM((nch, ct), jnp.int32),
            pltpu.VMEM((nch, ct), jnp.int32),
            pltpu.VMEM((ct, d), jnp.float32),
            pltpu.VMEM((ct, d), jnp.float32),
            pltpu.VMEM((ct, d), jnp.float32),
            pltpu.SemaphoreType.DMA,
        ],
    )
    def k(y_hbm, p0_hbm, p1_hbm, out_hbm, i0, i1, r0, r1, ob, sem):
        c = lax.axis_index("c")
        s = lax.axis_index("s")
        wid = s * NC + c
        pltpu.sync_copy(p0_hbm.at[wid], i0)
        pltpu.sync_copy(p1_hbm.at[wid], i1)
        base = wid * tok_w

        def cbody(ci, _):
            a = pltpu.async_copy(y_hbm.at[i0.at[ci]], r0, sem)
            b = pltpu.async_copy(y_hbm.at[i1.at[ci]], r1, sem)
            a.wait()
            b.wait()

            def tbody(ti, _):
                for j in range(d // LN):
                    sl = pl.ds(j * LN, LN)
                    ob[ti, sl] = r0[ti, sl] + r1[ti, sl]
                return 0
            lax.fori_loop(0, ct, tbody, 0)
            pltpu.sync_copy(ob, out_hbm.at[pl.ds(base + ci * ct, ct)])
            return 0
        lax.fori_loop(0, nch, cbody, 0)

    return k(y, pos0, pos1)


def kernel(inputs, gate_W, gate_b, expert_W, expert_b):
    t, d = inputs.shape
    ne = expert_W.shape[0]
    tk = 2 * t
    pad_t = tk + ne * TM
    nt = pad_t // TM
    nw = NC * NS

    eid, wts = _gate(inputs, gate_W, gate_b)
    ek = eid.T.reshape(-1)
    wk = wts.T.reshape(-1)
    tok_sorted, w_sorted, pos_k, eot = _routing(ek, wk, ne, pad_t, nt)
    tok3 = tok_sorted.reshape(nw, (pad_t // nw) // 32, 32)
    xs = _gather(inputs, tok3, pad_t)
    y = _grouped_mm(eot[:nt], xs, w_sorted, expert_W, expert_b, nt)
    pos0 = pos_k[:t].reshape(nw, (t // nw) // 16, 16)
    pos1 = pos_k[t:].reshape(nw, (t // nw) // 16, 16)
    return _combine(y, pos0, pos1, t, d)


# speedup vs baseline: 1.0381x; 1.0381x over previous
import functools

import jax
import jax.numpy as jnp
from jax import lax
from jax.experimental import pallas as pl
from jax.experimental.pallas import tpu as pltpu
from jax.experimental.pallas import tpu_sc as plsc

NC = 2
NS = 16
LN = 16
TM = 256


def _splat(vec, l):
    return lax.gather(
        vec, jnp.full((LN, 1), l, jnp.int32),
        lax.GatherDimensionNumbers(
            offset_dims=(), collapsed_slice_dims=(0,), start_index_map=(0,)),
        (1,), mode=lax.GatherScatterMode.PROMISE_IN_BOUNDS)


def _gate_body(x_ref, gwt_ref, gb_ref, eid_ref, w_ref):
    x = x_ref[...]
    logits = lax.dot_general(
        x, gwt_ref[...], (((1,), (0,)), ((), ())),
        preferred_element_type=jnp.float32) + gb_ref[...]
    ne = logits.shape[1]
    lanes = lax.broadcasted_iota(jnp.int32, logits.shape, 1)
    m1 = jnp.max(logits, axis=1, keepdims=True)
    idx1 = jnp.min(jnp.where(logits == m1, lanes, ne), axis=1, keepdims=True)
    l2 = jnp.where(lanes == idx1, -1e30, logits)
    m2 = jnp.max(l2, axis=1, keepdims=True)
    idx2 = jnp.min(jnp.where(l2 == m2, lanes, ne), axis=1, keepdims=True)
    e21 = jnp.exp(m2 - m1)
    w1 = 1.0 / (1.0 + e21)
    w2 = e21 / (1.0 + e21)
    eid_ref[...] = jnp.concatenate([idx1, idx2], axis=1)
    w_ref[...] = jnp.concatenate([w1, w2], axis=1)


def _gate(inputs, gate_W, gate_b):
    t, d = inputs.shape
    ne = gate_W.shape[0]
    bm = 512
    return pl.pallas_call(
        _gate_body,
        grid=(t // bm,),
        in_specs=[
            pl.BlockSpec((bm, d), lambda m: (m, 0)),
            pl.BlockSpec((d, ne), lambda m: (0, 0)),
            pl.BlockSpec((1, ne), lambda m: (0, 0)),
        ],
        out_specs=[
            pl.BlockSpec((bm, 2), lambda m: (m, 0)),
            pl.BlockSpec((bm, 2), lambda m: (m, 0)),
        ],
        out_shape=[
            jax.ShapeDtypeStruct((t, 2), jnp.int32),
            jax.ShapeDtypeStruct((t, 2), jnp.float32),
        ],
        compiler_params=pltpu.CompilerParams(
            dimension_semantics=("parallel",)),
    )(inputs, gate_W.T, gate_b.reshape(1, ne))


def _routing(ek, wk, ne, pad_t, nt):
    tk = ek.shape[0]
    tmask = tk // 2 - 1
    pp = tk // NS
    ch = pp // LN
    mesh = plsc.VectorSubcoreMesh(
        core_axis_name="c", subcore_axis_name="s",
        num_cores=NC, num_subcores=NS)
    cp = pltpu.CompilerParams(needs_layout_passes=False)

    @functools.partial(
        pl.kernel,
        out_type=[
            jax.ShapeDtypeStruct((NS, LN), jnp.int32),
            jax.ShapeDtypeStruct((NS, pp), jnp.int32),
        ],
        mesh=mesh,
        compiler_params=cp,
        scratch_types=[
            pltpu.VMEM((pp,), jnp.int32),
            pltpu.VMEM((pp,), jnp.int32),
            pltpu.VMEM((LN,), jnp.int32),
        ],
    )
    def r1(ek_hbm, cnt_out, lr_out, ev, lr, crow):
        c = lax.axis_index("c")
        s = lax.axis_index("s")

        @pl.when(c == 0)
        def _():
            lane16 = lax.broadcasted_iota(jnp.int32, (LN,), 0)
            base = s * pp
            pltpu.sync_copy(ek_hbm.at[pl.ds(base, pp)], ev)
            zero16 = jnp.zeros((LN,), jnp.int32)

            def cbody(i, cnts):
                evc = ev[pl.ds(i * LN, LN)]
                lrv = zero16
                new = []
                for e in range(ne):
                    m = evc == e
                    r = plsc.cumsum(m.astype(jnp.int32))
                    lrv = lrv + jnp.where(m, cnts[e] + r - 1, 0)
                    new.append(cnts[e] + _splat(r, LN - 1))
                lr[pl.ds(i * LN, LN)] = lrv
                return tuple(new)
            cnts = lax.fori_loop(
                0, ch, cbody, tuple(zero16 for _ in range(ne)))
            cv = zero16
            for e in range(ne):
                cv = cv + jnp.where(lane16 == e, cnts[e], 0)
            crow[pl.ds(0, LN)] = cv
            pltpu.sync_copy(crow, cnt_out.at[s])
            pltpu.sync_copy(lr, lr_out.at[s])

    cnt_tbl, lranks = r1(ek)

    @functools.partial(
        pl.kernel,
        out_type=[
            jax.ShapeDtypeStruct((NS, pp), jnp.int32),
            jax.ShapeDtypeStruct((80,), jnp.int32),
        ],
        mesh=mesh,
        compiler_params=cp,
        scratch_types=[
            pltpu.VMEM((pp,), jnp.int32),
            pltpu.VMEM((pp,), jnp.int32),
            pltpu.VMEM((pp,), jnp.int32),
            pltpu.VMEM((NS, LN), jnp.int32),
            pltpu.VMEM((80,), jnp.int32),
        ],
    )
    def r2(ek_hbm, cnt_hbm, lr_hbm, pos_out, eot_out,
           ev, lr, posb, tblv, eotb):
        c = lax.axis_index("c")
        s = lax.axis_index("s")

        @pl.when(c == 0)
        def _():
            base = s * pp
            pltpu.sync_copy(ek_hbm.at[pl.ds(base, pp)], ev)
            pltpu.sync_copy(lr_hbm.at[s], lr)
            pltpu.sync_copy(cnt_hbm, tblv)
            zero16 = jnp.zeros((LN,), jnp.int32)

            def rbody(ti, carry):
                tot, pre = carry
                row = tblv[ti]
                return tot + row, pre + row * (ti < s).astype(jnp.int32)
            tot, pre = lax.fori_loop(0, NS, rbody, (zero16, zero16))
            rounded = ((tot + (TM - 1)) >> 8) << 8
            incl = plsc.cumsum(rounded)
            startv = incl - rounded
            mybase = startv + pre
            mb = tuple(_splat(mybase, e) for e in range(ne))

            def cbody(i, _):
                evc = ev[pl.ds(i * LN, LN)]
                lrv = lr[pl.ds(i * LN, LN)]
                posv = zero16
                for e in range(ne):
                    posv = posv + jnp.where(evc == e, mb[e] + lrv, 0)
                posb[pl.ds(i * LN, LN)] = posv
                return 0
            lax.fori_loop(0, ch, cbody, 0)
            pltpu.sync_copy(posb, pos_out.at[s])

            @pl.when(s == 0)
            def _eot():
                lane16 = lax.broadcasted_iota(jnp.int32, (LN,), 0)
                inls = tuple(_splat(incl, e) for e in range(ne))

                def ebody(ci, _):
                    tvec = (lane16 + ci * LN) * TM
                    acc = jnp.zeros((LN,), jnp.int32)
                    for e in range(ne):
                        acc = acc + (tvec >= inls[e]).astype(jnp.int32)
                    eotb[pl.ds(ci * LN, LN)] = jnp.minimum(acc, ne - 1)
                    return 0
                lax.fori_loop(0, 80 // LN, ebody, 0)
                pltpu.sync_copy(eotb, eot_out)

    pos2d, eot = r2(ek, cnt_tbl, lranks)

    @functools.partial(
        pl.kernel,
        out_type=[
            jax.ShapeDtypeStruct((pad_t,), jnp.int32),
            jax.ShapeDtypeStruct((pad_t,), jnp.float32),
        ],
        mesh=mesh,
        compiler_params=cp,
        scratch_types=[
            pltpu.VMEM((tk,), jnp.int32),
            pltpu.VMEM((tk,), jnp.float32),
            pltpu.VMEM((pad_t,), jnp.int32),
            pltpu.VMEM((pad_t,), jnp.float32),
        ],
    )
    def r3(pos_hbm, wk_hbm, tok_out, w_out, pv, wv, tokb, wsb):
        c = lax.axis_index("c")
        s = lax.axis_index("s")

        @pl.when(jnp.logical_and(c == 0, s == 0))
        def _():
            lane16 = lax.broadcasted_iota(jnp.int32, (LN,), 0)
            pltpu.sync_copy(pos_hbm, pv)
            pltpu.sync_copy(wk_hbm, wv)

            def zb(i, _):
                tokb[pl.ds(i * LN, LN)] = jnp.zeros((LN,), jnp.int32)
                wsb[pl.ds(i * LN, LN)] = jnp.zeros((LN,), jnp.float32)
                return 0
            lax.fori_loop(0, pad_t // LN, zb, 0)

            def cb(i, _):
                posv = pv[pl.ds(i * LN, LN)]
                tokv = (jnp.full((LN,), i * LN, jnp.int32)
                        + lane16) & tmask
                plsc.store_scatter(tokb, [posv], tokv)
                plsc.store_scatter(wsb, [posv], wv[pl.ds(i * LN, LN)])
                return 0
            lax.fori_loop(0, tk // LN, cb, 0)
            pltpu.sync_copy(tokb, tok_out)
            pltpu.sync_copy(wsb, w_out)

    tok_sorted, w_sorted = r3(pos2d.reshape(-1), wk)
    return tok_sorted, w_sorted, pos2d.reshape(-1), eot


def _gather(inputs, tok3, pad_t):
    t, d = inputs.shape
    nw = NC * NS
    rows_w = pad_t // nw
    cr = 24
    nch = rows_w // cr
    nbuf = 4
    mesh = plsc.VectorSubcoreMesh(
        core_axis_name="c", subcore_axis_name="s",
        num_cores=NC, num_subcores=NS)

    @functools.partial(
        pl.kernel,
        out_type=jax.ShapeDtypeStruct((pad_t, d), jnp.float32),
        mesh=mesh,
        compiler_params=pltpu.CompilerParams(needs_layout_passes=False),
        scratch_types=[
            pltpu.VMEM((nch, cr), jnp.int32),
            pltpu.VMEM((nbuf, cr, d), jnp.float32),
            pltpu.SemaphoreType.DMA,
            pltpu.SemaphoreType.DMA,
            pltpu.SemaphoreType.DMA,
            pltpu.SemaphoreType.DMA,
            pltpu.SemaphoreType.DMA,
            pltpu.SemaphoreType.DMA,
            pltpu.SemaphoreType.DMA,
            pltpu.SemaphoreType.DMA,
        ],
    )
    def k(x_hbm, tok_hbm, xs_hbm, idx_v, bufs,
          g0, g1, g2, g3, w0, w1, w2, w3):
        c = lax.axis_index("c")
        s = lax.axis_index("s")
        wid = s * NC + c
        pltpu.sync_copy(tok_hbm.at[wid], idx_v)
        base = wid * rows_w
        gsem = (g0, g1, g2, g3)
        wsem = (w0, w1, w2, w3)
        gops = [None] * nch
        wops = [None] * nch
        for ci in range(min(nbuf - 1, nch)):
            gops[ci] = pltpu.async_copy(
                x_hbm.at[idx_v.at[ci]], bufs.at[ci % nbuf], gsem[ci % nbuf])
        for ci in range(nch):
            b = ci % nbuf
            if ci + nbuf - 1 < nch:
                if ci >= 1:
                    wops[ci - 1].wait()
                nb = (ci + nbuf - 1) % nbuf
                gops[ci + nbuf - 1] = pltpu.async_copy(
                    x_hbm.at[idx_v.at[ci + nbuf - 1]], bufs.at[nb], gsem[nb])
            gops[ci].wait()
            wops[ci] = pltpu.async_copy(
                bufs.at[b], xs_hbm.at[pl.ds(base + ci * cr, cr)], wsem[b])
        for ci in range(max(0, nch - nbuf), nch):
            if wops[ci] is not None and ci >= nch - nbuf:
                wops[ci].wait()
        return

    return k(inputs, tok3)


def _mm_body(eot_ref, xs_ref, ws_ref, ew_ref, eb_ref, y_ref):
    x = xs_ref[...].astype(jnp.bfloat16)
    w = ew_ref[0].astype(jnp.bfloat16)
    acc = lax.dot_general(
        x, w, (((1,), (1,)), ((), ())),
        preferred_element_type=jnp.float32)
    y_ref[...] = ws_ref[...] * (acc + eb_ref[0])


def _grouped_mm(eot, xs, ws, expert_W, expert_b, nt):
    ne, d, _ = expert_W.shape
    grid_spec = pltpu.PrefetchScalarGridSpec(
        num_scalar_prefetch=1,
        grid=(nt,),
        in_specs=[
            pl.BlockSpec((TM, d), lambda i, eot: (i, 0)),
            pl.BlockSpec((TM, 1), lambda i, eot: (i, 0)),
            pl.BlockSpec((1, d, d), lambda i, eot: (eot[i], 0, 0)),
            pl.BlockSpec((1, 1, d), lambda i, eot: (eot[i], 0, 0)),
        ],
        out_specs=pl.BlockSpec((TM, d), lambda i, eot: (i, 0)),
    )
    return pl.pallas_call(
        _mm_body,
        grid_spec=grid_spec,
        out_shape=jax.ShapeDtypeStruct((nt * TM, d), jnp.float32),
        compiler_params=pltpu.CompilerParams(
            dimension_semantics=("arbitrary",)),
    )(eot, xs, ws.reshape(-1, 1), expert_W, expert_b.reshape(ne, 1, d))


def _combine(y, pos0, pos1, t, d):
    nw = NC * NS
    tok_w = t // nw
    ct = 16
    nch = tok_w // ct
    mesh = plsc.VectorSubcoreMesh(
        core_axis_name="c", subcore_axis_name="s",
        num_cores=NC, num_subcores=NS)

    @functools.partial(
        pl.kernel,
        out_type=jax.ShapeDtypeStruct((t, d), jnp.float32),
        mesh=mesh,
        compiler_params=pltpu.CompilerParams(needs_layout_passes=False),
        scratch_types=[
            pltpu.VMEM((nch, ct), jnp.int32),
            pltpu.VMEM((nch, ct), jnp.int32),
            pltpu.VMEM((2, ct, d), jnp.float32),
            pltpu.VMEM((2, ct, d), jnp.float32),
            pltpu.VMEM((2, ct, d), jnp.float32),
            pltpu.SemaphoreType.DMA,
            pltpu.SemaphoreType.DMA,
            pltpu.SemaphoreType.DMA,
            pltpu.SemaphoreType.DMA,
        ],
    )
    def k(y_hbm, p0_hbm, p1_hbm, out_hbm, i0, i1, r0, r1, ob,
          gs0, gs1, ws0, ws1):
        c = lax.axis_index("c")
        s = lax.axis_index("s")
        wid = s * NC + c
        pltpu.sync_copy(p0_hbm.at[wid], i0)
        pltpu.sync_copy(p1_hbm.at[wid], i1)
        base = wid * tok_w
        gsem = (gs0, gs1)
        wsem = (ws0, ws1)
        gops = [None] * nch
        wops = [None] * nch

        def gissue(ci):
            b = ci % 2
            a1 = pltpu.async_copy(y_hbm.at[i0.at[ci]], r0.at[b], gsem[b])
            a2 = pltpu.async_copy(y_hbm.at[i1.at[ci]], r1.at[b], gsem[b])
            return (a1, a2)

        gops[0] = gissue(0)
        for ci in range(nch):
            b = ci % 2
            if ci + 1 < nch:
                gops[ci + 1] = gissue(ci + 1)
            gops[ci][0].wait()
            gops[ci][1].wait()
            if ci >= 2:
                wops[ci - 2].wait()

            def tbody(ti, _):
                for j in range(d // LN):
                    sl = pl.ds(j * LN, LN)
                    ob[b, ti, sl] = r0[b, ti, sl] + r1[b, ti, sl]
                return 0
            lax.fori_loop(0, ct, tbody, 0)
            wops[ci] = pltpu.async_copy(
                ob.at[b], out_hbm.at[pl.ds(base + ci * ct, ct)], wsem[b])
        wops[nch - 2].wait()
        wops[nch - 1].wait()

    return k(y, pos0, pos1)


def kernel(inputs, gate_W, gate_b, expert_W, expert_b):
    t, d = inputs.shape
    ne = expert_W.shape[0]
    tk = 2 * t
    pad_t = tk + ne * TM
    nt = pad_t // TM
    nw = NC * NS

    eid, wts = _gate(inputs, gate_W, gate_b)
    ek = eid.T.reshape(-1)
    wk = wts.T.reshape(-1)
    tok_sorted, w_sorted, pos_k, eot = _routing(ek, wk, ne, pad_t, nt)
    tok3 = tok_sorted.reshape(nw, (pad_t // nw) // 24, 24)
    xs = _gather(inputs, tok3, pad_t)
    y = _grouped_mm(eot[:nt], xs, w_sorted, expert_W, expert_b, nt)
    pos0 = pos_k[:t].reshape(nw, (t // nw) // 16, 16)
    pos1 = pos_k[t:].reshape(nw, (t // nw) // 16, 16)
    return _combine(y, pos0, pos1, t, d)
